# baseline (speedup 1.0000x reference)
import functools

import jax
import jax.numpy as jnp
from jax import lax
from jax.experimental import pallas as pl
from jax.experimental.pallas import tpu as pltpu
from jax.experimental.pallas import tpu_sc as plsc

N = 10000
E = 320000
D_IN = 128
D_H = 256
N_CLS = 40

NP = 10240
K = 128
NC = 2
NS = 16
NW = NC * NS
CPW = 80
NCHUNK = NW * CPW
EP = NCHUNK * K
RT = NP // NS

_mesh = plsc.VectorSubcoreMesh(core_axis_name="c", subcore_axis_name="s")


def _sc_degrees(src2d, dst2d, zeros_np):
    @functools.partial(
        pl.kernel,
        out_type=jax.ShapeDtypeStruct((NC, 2 * NP), jnp.float32),
        mesh=_mesh,
        scratch_types=[
            pltpu.VMEM((CPW, K), jnp.int32),
            pltpu.VMEM((CPW, K), jnp.int32),
            pltpu.VMEM((K,), jnp.float32),
            pltpu.VMEM_SHARED((NP,), jnp.float32),
            pltpu.VMEM_SHARED((NP,), jnp.float32),
        ],
    )
    def deg_kernel(src_hbm, dst_hbm, zeros_hbm, out_hbm,
                   src_v, dst_v, ones_v, do_sh, di_sh):
        cid = lax.axis_index("c")
        sid = lax.axis_index("s")
        wid = sid * NC + cid
        for i in range(K // 16):
            ones_v[pl.ds(i * 16, 16)] = jnp.ones((16,), jnp.float32)
        r0 = sid * RT
        pltpu.sync_copy(zeros_hbm.at[pl.ds(r0, RT)], do_sh.at[pl.ds(r0, RT)])
        pltpu.sync_copy(zeros_hbm.at[pl.ds(r0, RT)], di_sh.at[pl.ds(r0, RT)])
        base = wid * CPW
        pltpu.sync_copy(src_hbm.at[pl.ds(base, CPW)], src_v)
        pltpu.sync_copy(dst_hbm.at[pl.ds(base, CPW)], dst_v)
        plsc.subcore_barrier()

        def body(j, carry):
            pltpu.sync_copy(ones_v, do_sh.at[src_v.at[j]], add=True)
            pltpu.sync_copy(ones_v, di_sh.at[dst_v.at[j]], add=True)
            return carry

        lax.fori_loop(0, CPW, body, 0)
        plsc.subcore_barrier()
        pltpu.sync_copy(do_sh.at[pl.ds(r0, RT)],
                        out_hbm.at[cid, pl.ds(r0, RT)])
        pltpu.sync_copy(di_sh.at[pl.ds(r0, RT)],
                        out_hbm.at[cid, pl.ds(NP + r0, RT)])

    return deg_kernel(src2d, dst2d, zeros_np)


NBUF = 2
NRING = 4


def _sc_aggregate(t, src2d, dst2d, c0):
    D = t.shape[1]
    c1 = NCHUNK // NS - c0
    assert c0 % NRING == 0 and c1 % NRING == 0 and c1 >= NRING

    @functools.partial(
        pl.kernel,
        out_type=jax.ShapeDtypeStruct((NC, NP, D), jnp.float32),
        mesh=_mesh,
        compiler_params=pltpu.CompilerParams(use_tc_tiling_on_sc=False),
        scratch_types=[
            pltpu.VMEM((NRING, K), jnp.int32),
            pltpu.VMEM((NRING, K), jnp.int32),
            pltpu.VMEM((NBUF, K, D), jnp.float32),
            pltpu.VMEM_SHARED((NP, D), jnp.float32),
        ] + [pltpu.SemaphoreType.DMA] * (NBUF + 2 * NRING),
    )
    def agg_kernel(t_hbm, src_hbm, dst_hbm, out_hbm,
                   sring, dring, rows_v, acc_sh, *sems):
        gs = sems[:NBUF]
        isem = sems[NBUF:NBUF + NRING]
        dsem = sems[NBUF + NRING:]
        cid = lax.axis_index("c")
        sid = lax.axis_index("s")
        r0 = sid * RT
        ct = jnp.where(cid == 0, c0, c1)
        base = jnp.where(cid == 0, sid * c0, NS * c0 + sid * c1)

        offs = list(range(0, 16 * (D // 16), 16))
        if D % 16:
            offs.append(D - 16)

        def zb(r, carry):
            for off in offs:
                rows_v[0, r, pl.ds(off, 16)] = jnp.zeros((16,), jnp.float32)
            return carry

        lax.fori_loop(0, K, zb, 0)
        for q in range(RT // K):
            pltpu.sync_copy(rows_v.at[0], acc_sh.at[pl.ds(r0 + q * K, K)])

        for r in range(NRING):
            pltpu.async_copy(src_hbm.at[base + r], sring.at[r], isem[r])
            pltpu.async_copy(dst_hbm.at[base + r], dring.at[r], dsem[r])
        plsc.subcore_barrier()

        for m in range(NBUF):
            pltpu.make_async_copy(src_hbm.at[base + m], sring.at[m],
                                  isem[m]).wait()
            pltpu.async_copy(t_hbm.at[sring.at[m]], rows_v.at[m], gs[m])

        def body(i, carry):
            j0 = i * NRING
            for s in range(NRING):
                j = j0 + s
                rb = s % NBUF
                pltpu.make_async_copy(t_hbm.at[sring.at[s]],
                                      rows_v.at[rb], gs[rb]).wait()
                pltpu.make_async_copy(dst_hbm.at[base + j], dring.at[s],
                                      dsem[s]).wait()
                pltpu.sync_copy(rows_v.at[rb], acc_sh.at[dring.at[s]],
                                add=True)

                @pl.when(j + NBUF < ct)
                def _(s=s, j=j, rb=rb):
                    rn = (s + NBUF) % NRING
                    pltpu.make_async_copy(src_hbm.at[base + j + NBUF],
                                          sring.at[rn], isem[rn]).wait()
                    pltpu.async_copy(t_hbm.at[sring.at[rn]],
                                     rows_v.at[rb], gs[rb])

                @pl.when(j + NRING < ct)
                def _(s=s, j=j):
                    pltpu.async_copy(src_hbm.at[base + j + NRING],
                                     sring.at[s], isem[s])
                    pltpu.async_copy(dst_hbm.at[base + j + NRING],
                                     dring.at[s], dsem[s])
            return carry

        lax.fori_loop(0, ct // NRING, body, 0)
        plsc.subcore_barrier()
        pltpu.sync_copy(acc_sh.at[pl.ds(r0, RT)],
                        out_hbm.at[cid, pl.ds(r0, RT)])

    return agg_kernel(t, src2d, dst2d)


def _tc_prescale(x, deg4):
    def body(x_ref, deg_ref, t_ref, ns_ref, nd_ref):
        d = deg_ref[...]
        d_out = d[0, :N] + d[2, :N]
        d_in = d[1, :N] + d[3, :N]
        ns = jnp.where(d_out > 0, lax.rsqrt(jnp.maximum(d_out, 1e-12)), 0.0)
        nd = jnp.where(d_in > 0, lax.rsqrt(jnp.maximum(d_in, 1e-12)), 0.0)
        t_ref[:N, :] = x_ref[...] * ns[:, None]
        t_ref[N:, :] = jnp.zeros((NP - N, D_IN), jnp.float32)
        ns_ref[:N, :] = ns[:, None]
        ns_ref[N:, :] = jnp.zeros((NP - N, 1), jnp.float32)
        nd_ref[:N, :] = nd[:, None]
        nd_ref[N:, :] = jnp.zeros((NP - N, 1), jnp.float32)

    return pl.pallas_call(
        body,
        out_shape=[
            jax.ShapeDtypeStruct((NP, D_IN), jnp.float32),
            jax.ShapeDtypeStruct((NP, 1), jnp.float32),
            jax.ShapeDtypeStruct((NP, 1), jnp.float32),
        ],
    )(x, deg4)


_BR = 1024


def _tc_dense(u, ns, nd, W1, b1, W2):
    def body(u_ref, ns_ref, nd_ref, w1_ref, b1_ref, w2_ref, z_ref):
        agg = (u_ref[0] + u_ref[1]) * nd_ref[...]
        h = jnp.dot(agg, w1_ref[...], preferred_element_type=jnp.float32)
        h = jnp.maximum(h + b1_ref[...], 0.0) * ns_ref[...]
        z_ref[...] = jnp.dot(h, w2_ref[...],
                             preferred_element_type=jnp.float32)

    return pl.pallas_call(
        body,
        grid=(NP // _BR,),
        in_specs=[
            pl.BlockSpec((NC, _BR, D_IN), lambda i: (0, i, 0)),
            pl.BlockSpec((_BR, 1), lambda i: (i, 0)),
            pl.BlockSpec((_BR, 1), lambda i: (i, 0)),
            pl.BlockSpec((D_IN, D_H), lambda i: (0, 0)),
            pl.BlockSpec((1, D_H), lambda i: (0, 0)),
            pl.BlockSpec((D_H, N_CLS), lambda i: (0, 0)),
        ],
        out_specs=pl.BlockSpec((_BR, N_CLS), lambda i: (i, 0)),
        out_shape=jax.ShapeDtypeStruct((NP, N_CLS), jnp.float32),
    )(u, ns, nd, W1, b1, W2)


def _tc_final(v, nd, b2):
    def body(v_ref, nd_ref, b2_ref, o_ref):
        o_ref[...] = (v_ref[0] + v_ref[1]) * nd_ref[...] + b2_ref[...]

    BR = 1000
    return pl.pallas_call(
        body,
        grid=(N // BR,),
        in_specs=[
            pl.BlockSpec((NC, BR, N_CLS), lambda i: (0, i, 0)),
            pl.BlockSpec((BR, 1), lambda i: (i, 0)),
            pl.BlockSpec((1, N_CLS), lambda i: (0, 0)),
        ],
        out_specs=pl.BlockSpec((BR, N_CLS), lambda i: (i, 0)),
        out_shape=jax.ShapeDtypeStruct((N, N_CLS), jnp.float32),
    )(v, nd, b2)


def kernel(x, edge_index, W1, b1, W2, b2):
    pad = jnp.full((2, EP - E), N, dtype=jnp.int32)
    ei = jnp.concatenate([edge_index, pad], axis=1)
    src2d = ei[0].reshape(NCHUNK, K)
    dst2d = ei[1].reshape(NCHUNK, K)
    zeros1 = jnp.zeros((NP,), jnp.float32)

    deg = _sc_degrees(src2d, dst2d, zeros1)
    deg4 = deg.reshape(NC * 2, NP)
    t, ns, nd = _tc_prescale(x, deg4)
    u = _sc_aggregate(t, src2d, dst2d, 120)
    z = _tc_dense(u, ns, nd, W1, b1.reshape(1, D_H), W2)
    v = _sc_aggregate(z, src2d, dst2d, 112)
    return _tc_final(v, nd, b2.reshape(1, N_CLS))


# speedup vs baseline: 1.0079x; 1.0079x over previous
import functools

import jax
import jax.numpy as jnp
from jax import lax
from jax.experimental import pallas as pl
from jax.experimental.pallas import tpu as pltpu
from jax.experimental.pallas import tpu_sc as plsc

N = 10000
E = 320000
D_IN = 128
D_H = 256
N_CLS = 40

NP = 10240
K = 128
NC = 2
NS = 16
NW = NC * NS
CPW = 80
NCHUNK = NW * CPW
EP = NCHUNK * K
RT = NP // NS

_mesh = plsc.VectorSubcoreMesh(core_axis_name="c", subcore_axis_name="s")


def _sc_degrees(src2d, dst2d, zeros_np):
    @functools.partial(
        pl.kernel,
        out_type=jax.ShapeDtypeStruct((NC, 2 * NP), jnp.float32),
        mesh=_mesh,
        scratch_types=[
            pltpu.VMEM((CPW, K), jnp.int32),
            pltpu.VMEM((CPW, K), jnp.int32),
            pltpu.VMEM((K,), jnp.float32),
            pltpu.VMEM_SHARED((NP,), jnp.float32),
            pltpu.VMEM_SHARED((NP,), jnp.float32),
        ],
    )
    def deg_kernel(src_hbm, dst_hbm, zeros_hbm, out_hbm,
                   src_v, dst_v, ones_v, do_sh, di_sh):
        cid = lax.axis_index("c")
        sid = lax.axis_index("s")
        wid = sid * NC + cid
        for i in range(K // 16):
            ones_v[pl.ds(i * 16, 16)] = jnp.ones((16,), jnp.float32)
        r0 = sid * RT
        pltpu.sync_copy(zeros_hbm.at[pl.ds(r0, RT)], do_sh.at[pl.ds(r0, RT)])
        pltpu.sync_copy(zeros_hbm.at[pl.ds(r0, RT)], di_sh.at[pl.ds(r0, RT)])
        base = wid * CPW
        pltpu.sync_copy(src_hbm.at[pl.ds(base, CPW)], src_v)
        pltpu.sync_copy(dst_hbm.at[pl.ds(base, CPW)], dst_v)
        plsc.subcore_barrier()

        def body(j, carry):
            pltpu.sync_copy(ones_v, do_sh.at[src_v.at[j]], add=True)
            pltpu.sync_copy(ones_v, di_sh.at[dst_v.at[j]], add=True)
            return carry

        lax.fori_loop(0, CPW, body, 0)
        plsc.subcore_barrier()
        pltpu.sync_copy(do_sh.at[pl.ds(r0, RT)],
                        out_hbm.at[cid, pl.ds(r0, RT)])
        pltpu.sync_copy(di_sh.at[pl.ds(r0, RT)],
                        out_hbm.at[cid, pl.ds(NP + r0, RT)])

    return deg_kernel(src2d, dst2d, zeros_np)


NBUF = 2
NRING = 4


def _sc_aggregate(t, src2d, dst2d, c0):
    D = t.shape[1]
    c1 = NCHUNK // NS - c0
    assert c0 % NRING == 0 and c1 % NRING == 0 and c1 >= NRING
    dst_rows = max(c0, c1)
    words = NP * D + NS * (NRING * K + dst_rows * K + NBUF * K * D)
    dst_full = words <= 2_000_000
    if not dst_full:
        dst_rows = NRING

    @functools.partial(
        pl.kernel,
        out_type=jax.ShapeDtypeStruct((NC, NP, D), jnp.float32),
        mesh=_mesh,
        compiler_params=pltpu.CompilerParams(use_tc_tiling_on_sc=False),
        scratch_types=[
            pltpu.VMEM((NRING, K), jnp.int32),
            pltpu.VMEM((dst_rows, K), jnp.int32),
            pltpu.VMEM((NBUF, K, D), jnp.float32),
            pltpu.VMEM_SHARED((NP, D), jnp.float32),
        ] + [pltpu.SemaphoreType.DMA] * (NBUF + 2 * NRING),
    )
    def agg_kernel(t_hbm, src_hbm, dst_hbm, out_hbm,
                   sring, dring, rows_v, acc_sh, *sems):
        gs = sems[:NBUF]
        isem = sems[NBUF:NBUF + NRING]
        dsem = sems[NBUF + NRING:]
        cid = lax.axis_index("c")
        sid = lax.axis_index("s")
        r0 = sid * RT
        ct = jnp.where(cid == 0, c0, c1)
        base = jnp.where(cid == 0, sid * c0, NS * c0 + sid * c1)

        offs = list(range(0, 16 * (D // 16), 16))
        if D % 16:
            offs.append(D - 16)

        def zb(r, carry):
            for off in offs:
                rows_v[0, r, pl.ds(off, 16)] = jnp.zeros((16,), jnp.float32)
            return carry

        lax.fori_loop(0, K, zb, 0)
        for q in range(RT // K):
            pltpu.sync_copy(rows_v.at[0], acc_sh.at[pl.ds(r0 + q * K, K)])

        if dst_full:
            @pl.when(cid == 0)
            def _():
                pltpu.sync_copy(dst_hbm.at[pl.ds(sid * c0, c0)],
                                dring.at[pl.ds(0, c0)])

            @pl.when(cid == 1)
            def _():
                pltpu.sync_copy(dst_hbm.at[pl.ds(NS * c0 + sid * c1, c1)],
                                dring.at[pl.ds(0, c1)])

        for r in range(NRING):
            pltpu.async_copy(src_hbm.at[base + r], sring.at[r], isem[r])
            if not dst_full:
                pltpu.async_copy(dst_hbm.at[base + r], dring.at[r], dsem[r])
        plsc.subcore_barrier()

        for m in range(NBUF):
            pltpu.make_async_copy(src_hbm.at[base + m], sring.at[m],
                                  isem[m]).wait()
            pltpu.async_copy(t_hbm.at[sring.at[m]], rows_v.at[m], gs[m])

        def body(i, carry):
            j0 = i * NRING
            for s in range(NRING):
                j = j0 + s
                rb = s % NBUF
                pltpu.make_async_copy(t_hbm.at[sring.at[s]],
                                      rows_v.at[rb], gs[rb]).wait()
                if dst_full:
                    pltpu.sync_copy(rows_v.at[rb], acc_sh.at[dring.at[j]],
                                    add=True)
                else:
                    pltpu.make_async_copy(dst_hbm.at[base + j], dring.at[s],
                                          dsem[s]).wait()
                    pltpu.sync_copy(rows_v.at[rb], acc_sh.at[dring.at[s]],
                                    add=True)

                @pl.when(j + NBUF < ct)
                def _(s=s, j=j, rb=rb):
                    rn = (s + NBUF) % NRING
                    pltpu.make_async_copy(src_hbm.at[base + j + NBUF],
                                          sring.at[rn], isem[rn]).wait()
                    pltpu.async_copy(t_hbm.at[sring.at[rn]],
                                     rows_v.at[rb], gs[rb])

                @pl.when(j + NRING < ct)
                def _(s=s, j=j):
                    pltpu.async_copy(src_hbm.at[base + j + NRING],
                                     sring.at[s], isem[s])
                    if not dst_full:
                        pltpu.async_copy(dst_hbm.at[base + j + NRING],
                                         dring.at[s], dsem[s])
            return carry

        lax.fori_loop(0, ct // NRING, body, 0)
        plsc.subcore_barrier()
        pltpu.sync_copy(acc_sh.at[pl.ds(r0, RT)],
                        out_hbm.at[cid, pl.ds(r0, RT)])

    return agg_kernel(t, src2d, dst2d)


def _tc_prescale(x, deg4):
    def body(x_ref, deg_ref, t_ref, ns_ref, nd_ref):
        d = deg_ref[...]
        d_out = d[0, :N] + d[2, :N]
        d_in = d[1, :N] + d[3, :N]
        ns = jnp.where(d_out > 0, lax.rsqrt(jnp.maximum(d_out, 1e-12)), 0.0)
        nd = jnp.where(d_in > 0, lax.rsqrt(jnp.maximum(d_in, 1e-12)), 0.0)
        t_ref[:N, :] = x_ref[...] * ns[:, None]
        t_ref[N:, :] = jnp.zeros((NP - N, D_IN), jnp.float32)
        ns_ref[:N, :] = ns[:, None]
        ns_ref[N:, :] = jnp.zeros((NP - N, 1), jnp.float32)
        nd_ref[:N, :] = nd[:, None]
        nd_ref[N:, :] = jnp.zeros((NP - N, 1), jnp.float32)

    return pl.pallas_call(
        body,
        out_shape=[
            jax.ShapeDtypeStruct((NP, D_IN), jnp.float32),
            jax.ShapeDtypeStruct((NP, 1), jnp.float32),
            jax.ShapeDtypeStruct((NP, 1), jnp.float32),
        ],
    )(x, deg4)


_BR = 1024


def _tc_dense(u, ns, nd, W1, b1, W2):
    def body(u_ref, ns_ref, nd_ref, w1_ref, b1_ref, w2_ref, z_ref):
        agg = (u_ref[0] + u_ref[1]) * nd_ref[...]
        h = jnp.dot(agg, w1_ref[...], preferred_element_type=jnp.float32)
        h = jnp.maximum(h + b1_ref[...], 0.0) * ns_ref[...]
        z_ref[...] = jnp.dot(h, w2_ref[...],
                             preferred_element_type=jnp.float32)

    return pl.pallas_call(
        body,
        grid=(NP // _BR,),
        in_specs=[
            pl.BlockSpec((NC, _BR, D_IN), lambda i: (0, i, 0)),
            pl.BlockSpec((_BR, 1), lambda i: (i, 0)),
            pl.BlockSpec((_BR, 1), lambda i: (i, 0)),
            pl.BlockSpec((D_IN, D_H), lambda i: (0, 0)),
            pl.BlockSpec((1, D_H), lambda i: (0, 0)),
            pl.BlockSpec((D_H, N_CLS), lambda i: (0, 0)),
        ],
        out_specs=pl.BlockSpec((_BR, N_CLS), lambda i: (i, 0)),
        out_shape=jax.ShapeDtypeStruct((NP, N_CLS), jnp.float32),
    )(u, ns, nd, W1, b1, W2)


def _tc_final(v, nd, b2):
    def body(v_ref, nd_ref, b2_ref, o_ref):
        o_ref[...] = (v_ref[0] + v_ref[1]) * nd_ref[...] + b2_ref[...]

    BR = 1000
    return pl.pallas_call(
        body,
        grid=(N // BR,),
        in_specs=[
            pl.BlockSpec((NC, BR, N_CLS), lambda i: (0, i, 0)),
            pl.BlockSpec((BR, 1), lambda i: (i, 0)),
            pl.BlockSpec((1, N_CLS), lambda i: (0, 0)),
        ],
        out_specs=pl.BlockSpec((BR, N_CLS), lambda i: (i, 0)),
        out_shape=jax.ShapeDtypeStruct((N, N_CLS), jnp.float32),
    )(v, nd, b2)


def kernel(x, edge_index, W1, b1, W2, b2):
    pad = jnp.full((2, EP - E), N, dtype=jnp.int32)
    ei = jnp.concatenate([edge_index, pad], axis=1)
    src2d = ei[0].reshape(NCHUNK, K)
    dst2d = ei[1].reshape(NCHUNK, K)
    zeros1 = jnp.zeros((NP,), jnp.float32)

    deg = _sc_degrees(src2d, dst2d, zeros1)
    deg4 = deg.reshape(NC * 2, NP)
    t, ns, nd = _tc_prescale(x, deg4)
    u = _sc_aggregate(t, src2d, dst2d, 140)
    z = _tc_dense(u, ns, nd, W1, b1.reshape(1, D_H), W2)
    v = _sc_aggregate(z, src2d, dst2d, 132)
    return _tc_final(v, nd, b2.reshape(1, N_CLS))


# speedup vs baseline: 1.0322x; 1.0241x over previous
import functools

import jax
import jax.numpy as jnp
from jax import lax
from jax.experimental import pallas as pl
from jax.experimental.pallas import tpu as pltpu
from jax.experimental.pallas import tpu_sc as plsc

N = 10000
E = 320000
D_IN = 128
D_H = 256
N_CLS = 40

NP = 10240
K = 128
NC = 2
NS = 16
NW = NC * NS
CPW = 80
NCHUNK = NW * CPW
EP = NCHUNK * K
RT = NP // NS

NBUF = 2
NRING = 4

_mesh = plsc.VectorSubcoreMesh(core_axis_name="c", subcore_axis_name="s")


def _sc_degrees(src2d, dst2d, zeros_np):
    @functools.partial(
        pl.kernel,
        out_type=jax.ShapeDtypeStruct((NC, 2 * NP), jnp.float32),
        mesh=_mesh,
        scratch_types=[
            pltpu.VMEM((CPW, K), jnp.int32),
            pltpu.VMEM((CPW, K), jnp.int32),
            pltpu.VMEM((K,), jnp.float32),
            pltpu.VMEM_SHARED((NP,), jnp.float32),
            pltpu.VMEM_SHARED((NP,), jnp.float32),
        ],
    )
    def deg_kernel(src_hbm, dst_hbm, zeros_hbm, out_hbm,
                   src_v, dst_v, ones_v, do_sh, di_sh):
        cid = lax.axis_index("c")
        sid = lax.axis_index("s")
        wid = sid * NC + cid
        for i in range(K // 16):
            ones_v[pl.ds(i * 16, 16)] = jnp.ones((16,), jnp.float32)
        r0 = sid * RT
        pltpu.sync_copy(zeros_hbm.at[pl.ds(r0, RT)], do_sh.at[pl.ds(r0, RT)])
        pltpu.sync_copy(zeros_hbm.at[pl.ds(r0, RT)], di_sh.at[pl.ds(r0, RT)])
        base = wid * CPW
        pltpu.sync_copy(src_hbm.at[pl.ds(base, CPW)], src_v)
        pltpu.sync_copy(dst_hbm.at[pl.ds(base, CPW)], dst_v)
        plsc.subcore_barrier()

        def body(j, carry):
            pltpu.sync_copy(ones_v, do_sh.at[src_v.at[j]], add=True)
            pltpu.sync_copy(ones_v, di_sh.at[dst_v.at[j]], add=True)
            return carry

        lax.fori_loop(0, CPW, body, 0)
        plsc.subcore_barrier()
        pltpu.sync_copy(do_sh.at[pl.ds(r0, RT)],
                        out_hbm.at[cid, pl.ds(r0, RT)])
        pltpu.sync_copy(di_sh.at[pl.ds(r0, RT)],
                        out_hbm.at[cid, pl.ds(NP + r0, RT)])

    return deg_kernel(src2d, dst2d, zeros_np)


def _sc_aggregate(t, src2d, dst2d, zeros_nd, c0):
    D = t.shape[1]
    c1 = NCHUNK // NS - c0
    assert c0 % NRING == 0 and c1 % NRING == 0 and c1 >= NRING

    @functools.partial(
        pl.kernel,
        out_type=jax.ShapeDtypeStruct((NC, NP, D), jnp.float32),
        mesh=_mesh,
        compiler_params=pltpu.CompilerParams(use_tc_tiling_on_sc=False),
        scratch_types=[
            pltpu.VMEM((NRING, K), jnp.int32),
            pltpu.VMEM((max(c0, c1), K), jnp.int32),
            pltpu.VMEM((NBUF, K, D), jnp.float32),
            pltpu.VMEM_SHARED((NP, D), jnp.float32),
        ] + [pltpu.SemaphoreType.DMA] * (NBUF + NRING),
    )
    def agg_kernel(t_hbm, src_hbm, dst_hbm, zeros_hbm, out_hbm,
                   sring, dst_v, rows_v, acc_sh, *sems):
        gs = sems[:NBUF]
        isem = sems[NBUF:]
        cid = lax.axis_index("c")
        sid = lax.axis_index("s")
        r0 = sid * RT
        pltpu.sync_copy(zeros_hbm.at[pl.ds(r0, RT)], acc_sh.at[pl.ds(r0, RT)])
        ct = jnp.where(cid == 0, c0, c1)
        base = jnp.where(cid == 0, sid * c0, NS * c0 + sid * c1)

        @pl.when(cid == 0)
        def _():
            pltpu.sync_copy(dst_hbm.at[pl.ds(sid * c0, c0)],
                            dst_v.at[pl.ds(0, c0)])

        @pl.when(cid == 1)
        def _():
            pltpu.sync_copy(dst_hbm.at[pl.ds(NS * c0 + sid * c1, c1)],
                            dst_v.at[pl.ds(0, c1)])

        for r in range(NRING):
            pltpu.async_copy(src_hbm.at[base + r], sring.at[r], isem[r])
        plsc.subcore_barrier()

        for m in range(NBUF):
            pltpu.make_async_copy(src_hbm.at[base + m], sring.at[m],
                                  isem[m]).wait()
            pltpu.async_copy(t_hbm.at[sring.at[m]], rows_v.at[m], gs[m])

        def body(i, carry):
            j0 = i * NRING
            for s in range(NRING):
                j = j0 + s
                rb = s % NBUF
                pltpu.make_async_copy(t_hbm.at[sring.at[s]],
                                      rows_v.at[rb], gs[rb]).wait()
                pltpu.sync_copy(rows_v.at[rb], acc_sh.at[dst_v.at[j]],
                                add=True)

                @pl.when(j + NBUF < ct)
                def _(s=s, j=j, rb=rb):
                    rn = (s + NBUF) % NRING
                    pltpu.make_async_copy(src_hbm.at[base + j + NBUF],
                                          sring.at[rn], isem[rn]).wait()
                    pltpu.async_copy(t_hbm.at[sring.at[rn]],
                                     rows_v.at[rb], gs[rb])

                @pl.when(j + NRING < ct)
                def _(s=s, j=j):
                    pltpu.async_copy(src_hbm.at[base + j + NRING],
                                     sring.at[s], isem[s])
            return carry

        lax.fori_loop(0, ct // NRING, body, 0)
        plsc.subcore_barrier()
        pltpu.sync_copy(acc_sh.at[pl.ds(r0, RT)],
                        out_hbm.at[cid, pl.ds(r0, RT)])

    return agg_kernel(t, src2d, dst2d, zeros_nd)


def _tc_prescale(x, deg4):
    def body(x_ref, deg_ref, t_ref, ns_ref, nd_ref):
        d = deg_ref[...]
        d_out = d[0, :N] + d[2, :N]
        d_in = d[1, :N] + d[3, :N]
        ns = jnp.where(d_out > 0, lax.rsqrt(jnp.maximum(d_out, 1e-12)), 0.0)
        nd = jnp.where(d_in > 0, lax.rsqrt(jnp.maximum(d_in, 1e-12)), 0.0)
        t_ref[:N, :] = x_ref[...] * ns[:, None]
        t_ref[N:, :] = jnp.zeros((NP - N, D_IN), jnp.float32)
        ns_ref[:N, :] = ns[:, None]
        ns_ref[N:, :] = jnp.zeros((NP - N, 1), jnp.float32)
        nd_ref[:N, :] = nd[:, None]
        nd_ref[N:, :] = jnp.zeros((NP - N, 1), jnp.float32)

    return pl.pallas_call(
        body,
        out_shape=[
            jax.ShapeDtypeStruct((NP, D_IN), jnp.float32),
            jax.ShapeDtypeStruct((NP, 1), jnp.float32),
            jax.ShapeDtypeStruct((NP, 1), jnp.float32),
        ],
    )(x, deg4)


_BR = 1024


def _tc_dense(u, ns, nd, W1, b1, W2):
    def body(u_ref, ns_ref, nd_ref, w1_ref, b1_ref, w2_ref, z_ref):
        agg = (u_ref[0] + u_ref[1]) * nd_ref[...]
        h = jnp.dot(agg, w1_ref[...], preferred_element_type=jnp.float32)
        h = jnp.maximum(h + b1_ref[...], 0.0) * ns_ref[...]
        z_ref[...] = jnp.dot(h, w2_ref[...],
                             preferred_element_type=jnp.float32)

    return pl.pallas_call(
        body,
        grid=(NP // _BR,),
        in_specs=[
            pl.BlockSpec((NC, _BR, D_IN), lambda i: (0, i, 0)),
            pl.BlockSpec((_BR, 1), lambda i: (i, 0)),
            pl.BlockSpec((_BR, 1), lambda i: (i, 0)),
            pl.BlockSpec((D_IN, D_H), lambda i: (0, 0)),
            pl.BlockSpec((1, D_H), lambda i: (0, 0)),
            pl.BlockSpec((D_H, N_CLS), lambda i: (0, 0)),
        ],
        out_specs=pl.BlockSpec((_BR, N_CLS), lambda i: (i, 0)),
        out_shape=jax.ShapeDtypeStruct((NP, N_CLS), jnp.float32),
    )(u, ns, nd, W1, b1, W2)


def _tc_final(v, nd, b2):
    def body(v_ref, nd_ref, b2_ref, o_ref):
        o_ref[...] = (v_ref[0] + v_ref[1]) * nd_ref[...] + b2_ref[...]

    BR = 1000
    return pl.pallas_call(
        body,
        grid=(N // BR,),
        in_specs=[
            pl.BlockSpec((NC, BR, N_CLS), lambda i: (0, i, 0)),
            pl.BlockSpec((BR, 1), lambda i: (i, 0)),
            pl.BlockSpec((1, N_CLS), lambda i: (0, 0)),
        ],
        out_specs=pl.BlockSpec((BR, N_CLS), lambda i: (i, 0)),
        out_shape=jax.ShapeDtypeStruct((N, N_CLS), jnp.float32),
    )(v, nd, b2)


def kernel(x, edge_index, W1, b1, W2, b2):
    pad = jnp.full((2, EP - E), N, dtype=jnp.int32)
    ei = jnp.concatenate([edge_index, pad], axis=1)
    src2d = ei[0].reshape(NCHUNK, K)
    dst2d = ei[1].reshape(NCHUNK, K)
    zeros1 = jnp.zeros((NP,), jnp.float32)
    zeros_din = jnp.zeros((NP, D_IN), jnp.float32)
    zeros_cls = jnp.zeros((NP, N_CLS), jnp.float32)

    deg = _sc_degrees(src2d, dst2d, zeros1)
    deg4 = deg.reshape(NC * 2, NP)
    t, ns, nd = _tc_prescale(x, deg4)
    u = _sc_aggregate(t, src2d, dst2d, zeros_din, 120)
    z = _tc_dense(u, ns, nd, W1, b1.reshape(1, D_H), W2)
    v = _sc_aggregate(z, src2d, dst2d, zeros_cls, 112)
    return _tc_final(v, nd, b2.reshape(1, N_CLS))
